# baseline (speedup 1.0000x reference)
import functools

import jax
import jax.numpy as jnp
from jax import lax
from jax.experimental import pallas as pl
from jax.experimental.pallas import tpu as pltpu
from jax.experimental.pallas import tpu_sc as plsc

NUM_FIELDS = 26
VOCAB_PER_FIELD = 100000
EMBED_DIM = 128
BATCH = 4096

NC, NS, L = 2, 16, 16
NW = NC * NS
N_FLAT = BATCH * NUM_FIELDS
PER_W = N_FLAT // NW
UNIT = 256
UPW = PER_W // UNIT
UPF = BATCH // UNIT
CHUNK = 104
N_CHUNK = PER_W // CHUNK
NBUF = 8
AHEAD = 6


@functools.partial(
    pl.kernel,
    out_type=jax.ShapeDtypeStruct((N_FLAT, EMBED_DIM), jnp.float32),
    mesh=plsc.VectorSubcoreMesh(core_axis_name="c", subcore_axis_name="s"),
    scratch_types=[
        pltpu.VMEM((PER_W,), jnp.int32),
    ] + [pltpu.VMEM((CHUNK, EMBED_DIM), jnp.float32) for _ in range(NBUF)]
      + [pltpu.SemaphoreType.DMA for _ in range(2 * NBUF)]
      + [pltpu.SemaphoreType.DMA],
    compiler_params=pltpu.CompilerParams(
        disable_bounds_checks=True,
        disable_semaphore_checks=True,
    ),
)
def _sok_gather(idx_hbm, table_hbm, out_hbm, idx_v, *rest):
    bufs = rest[:NBUF]
    gsem = rest[NBUF:2 * NBUF]
    ssem = rest[2 * NBUF:3 * NBUF]
    isem = rest[3 * NBUF]
    wid = lax.axis_index("s") * NC + lax.axis_index("c")
    base = wid * PER_W

    for u in range(UPW):
        g = wid * UPW + u
        f = lax.div(g, UPF)
        c = lax.rem(g, UPF)
        pltpu.async_copy(idx_hbm.at[f, pl.ds(c * UNIT, UNIT)],
                         idx_v.at[pl.ds(u * UNIT, UNIT)], isem)
    for u in range(UPW):
        pltpu.make_async_copy(idx_hbm.at[0, pl.ds(0, UNIT)],
                              idx_v.at[pl.ds(0, UNIT)], isem).wait()
    iota = lax.iota(jnp.int32, L)

    def fuse(t):
        off = lax.div(wid * PER_W + t * L, BATCH) * VOCAB_PER_FIELD
        idx_v[pl.ds(t * L, L)] = idx_v[pl.ds(t * L, L)] + off

    FUSE1 = (AHEAD * CHUNK + L - 1) // L + 1

    @pl.loop(0, FUSE1, unroll=8)
    def _fuse_head(t):
        fuse(t)

    def gather(j, b):
        pltpu.async_copy(table_hbm.at[idx_v.at[pl.ds(j * CHUNK, CHUNK)]],
                         bufs[b], gsem[b])

    def wait_gather(b):
        pltpu.make_async_copy(table_hbm.at[pl.ds(0, CHUNK)], bufs[b],
                              gsem[b]).wait()

    def store(j, b):
        pltpu.async_copy(bufs[b], out_hbm.at[pl.ds(base + j * CHUNK, CHUNK)],
                         ssem[b])

    def wait_store(b):
        pltpu.make_async_copy(bufs[b], out_hbm.at[pl.ds(0, CHUNK)],
                              ssem[b]).wait()

    def step(j, b, prefetch, wait_prev):
        wait_gather(b)
        store(j, b)
        if prefetch:
            bn = (b + AHEAD) % NBUF
            if wait_prev:
                wait_store(bn)
            gather(j + AHEAD, bn)

    for b in range(AHEAD):
        gather(b, b)

    @pl.loop(FUSE1, PER_W // L, unroll=8)
    def _fuse_tail(t):
        fuse(t)

    for j in range(NBUF):
        step(j, j, True, j + AHEAD >= NBUF)

    @pl.loop(NBUF, N_CHUNK - NBUF, step=NBUF)
    def _main(j0):
        for bb in range(NBUF):
            step(j0 + bb, bb, True, True)

    for j in range(N_CHUNK - NBUF, N_CHUNK):
        step(j, j % NBUF, j + AHEAD < N_CHUNK, True)
    for j in range(N_CHUNK - NBUF, N_CHUNK):
        wait_store(j % NBUF)


def kernel(inputs, table):
    out = _sok_gather(inputs.T, table)
    return out.reshape(NUM_FIELDS, BATCH, EMBED_DIM).transpose(1, 0, 2)


# speedup vs baseline: 1.0036x; 1.0036x over previous
import functools

import jax
import jax.numpy as jnp
from jax import lax
from jax.experimental import pallas as pl
from jax.experimental.pallas import tpu as pltpu
from jax.experimental.pallas import tpu_sc as plsc

NUM_FIELDS = 26
VOCAB_PER_FIELD = 100000
EMBED_DIM = 128
BATCH = 4096

NC, NS, L = 2, 16, 16
NW = NC * NS
N_FLAT = BATCH * NUM_FIELDS
PER_W = N_FLAT // NW
UNIT = 256
UPW = PER_W // UNIT
UPF = BATCH // UNIT
CHUNK = 128
N_CHUNK = PER_W // CHUNK
NBUF = 7
AHEAD = 5


@functools.partial(
    pl.kernel,
    out_type=jax.ShapeDtypeStruct((N_FLAT, EMBED_DIM), jnp.float32),
    mesh=plsc.VectorSubcoreMesh(core_axis_name="c", subcore_axis_name="s"),
    scratch_types=[
        pltpu.VMEM((PER_W,), jnp.int32),
    ] + [pltpu.VMEM((CHUNK, EMBED_DIM), jnp.float32) for _ in range(NBUF)]
      + [pltpu.SemaphoreType.DMA for _ in range(2 * NBUF)]
      + [pltpu.SemaphoreType.DMA],
)
def _sok_gather(idx_hbm, table_hbm, out_hbm, idx_v, *rest):
    bufs = rest[:NBUF]
    gsem = rest[NBUF:2 * NBUF]
    ssem = rest[2 * NBUF:3 * NBUF]
    isem = rest[3 * NBUF]
    wid = lax.axis_index("s") * NC + lax.axis_index("c")
    base = wid * PER_W

    for u in range(UPW):
        g = wid * UPW + u
        f = lax.div(g, UPF)
        c = lax.rem(g, UPF)
        pltpu.async_copy(idx_hbm.at[f, pl.ds(c * UNIT, UNIT)],
                         idx_v.at[pl.ds(u * UNIT, UNIT)], isem)
    for u in range(UPW):
        pltpu.make_async_copy(idx_hbm.at[0, pl.ds(0, UNIT)],
                              idx_v.at[pl.ds(0, UNIT)], isem).wait()
    iota = lax.iota(jnp.int32, L)

    def fuse(t):
        off = lax.div(wid * PER_W + t * L, BATCH) * VOCAB_PER_FIELD
        idx_v[pl.ds(t * L, L)] = idx_v[pl.ds(t * L, L)] + off

    FUSE1 = (AHEAD * CHUNK + L - 1) // L + 1

    @pl.loop(0, FUSE1, unroll=8)
    def _fuse_head(t):
        fuse(t)

    def gather(j, b):
        pltpu.async_copy(table_hbm.at[idx_v.at[pl.ds(j * CHUNK, CHUNK)]],
                         bufs[b], gsem[b])

    def wait_gather(b):
        pltpu.make_async_copy(table_hbm.at[pl.ds(0, CHUNK)], bufs[b],
                              gsem[b]).wait()

    def store(j, b):
        pltpu.async_copy(bufs[b], out_hbm.at[pl.ds(base + j * CHUNK, CHUNK)],
                         ssem[b])

    def wait_store(b):
        pltpu.make_async_copy(bufs[b], out_hbm.at[pl.ds(0, CHUNK)],
                              ssem[b]).wait()

    def step(j, b, prefetch, wait_prev):
        wait_gather(b)
        store(j, b)
        if prefetch:
            bn = (b + AHEAD) % NBUF
            if wait_prev:
                wait_store(bn)
            gather(j + AHEAD, bn)

    for b in range(AHEAD):
        gather(b, b)

    @pl.loop(FUSE1, PER_W // L, unroll=8)
    def _fuse_tail(t):
        fuse(t)

    for j in range(N_CHUNK):
        step(j, j % NBUF, j + AHEAD < N_CHUNK,
             j + AHEAD >= NBUF)
    for j in range(N_CHUNK - NBUF, N_CHUNK):
        wait_store(j % NBUF)


def kernel(inputs, table):
    out = _sok_gather(inputs.T, table)
    return out.reshape(NUM_FIELDS, BATCH, EMBED_DIM).transpose(1, 0, 2)


# speedup vs baseline: 1.0058x; 1.0022x over previous
import functools

import jax
import jax.numpy as jnp
from jax import lax
from jax.experimental import pallas as pl
from jax.experimental.pallas import tpu as pltpu
from jax.experimental.pallas import tpu_sc as plsc

NUM_FIELDS = 26
VOCAB_PER_FIELD = 100000
EMBED_DIM = 128
BATCH = 4096

NC, NS, L = 2, 16, 16
NW = NC * NS
N_FLAT = BATCH * NUM_FIELDS
PER_W = N_FLAT // NW
UNIT = 256
UPW = PER_W // UNIT
UPF = BATCH // UNIT
CHUNK = 128
N_CHUNK = PER_W // CHUNK
NBUF = 7
AHEAD = 5


@functools.partial(
    pl.kernel,
    out_type=jax.ShapeDtypeStruct((N_FLAT, EMBED_DIM), jnp.float32),
    mesh=plsc.VectorSubcoreMesh(core_axis_name="c", subcore_axis_name="s"),
    scratch_types=[
        pltpu.VMEM((PER_W,), jnp.int32),
    ] + [pltpu.VMEM((CHUNK, EMBED_DIM), jnp.float32) for _ in range(NBUF)]
      + [pltpu.SemaphoreType.DMA for _ in range(2 * NBUF)]
      + [pltpu.SemaphoreType.DMA, pltpu.SemaphoreType.DMA],
)
def _sok_gather(idx_hbm, table_hbm, out_hbm, idx_v, *rest):
    bufs = rest[:NBUF]
    gsem = rest[NBUF:2 * NBUF]
    ssem = rest[2 * NBUF:3 * NBUF]
    isem = rest[3 * NBUF]
    isem_head = rest[3 * NBUF + 1]
    wid = lax.axis_index("s") * NC + lax.axis_index("c")
    base = wid * PER_W

    HEAD_UNITS = 3
    for u in range(UPW):
        g = wid * UPW + u
        f = lax.div(g, UPF)
        c = lax.rem(g, UPF)
        pltpu.async_copy(idx_hbm.at[f, pl.ds(c * UNIT, UNIT)],
                         idx_v.at[pl.ds(u * UNIT, UNIT)],
                         isem_head if u < HEAD_UNITS else isem)

    def wait_units(sem, n):
        for _ in range(n):
            pltpu.make_async_copy(idx_hbm.at[0, pl.ds(0, UNIT)],
                                  idx_v.at[pl.ds(0, UNIT)], sem).wait()

    wait_units(isem_head, HEAD_UNITS)
    iota = lax.iota(jnp.int32, L)

    def fuse(t):
        off = lax.div(wid * PER_W + t * L, BATCH) * VOCAB_PER_FIELD
        idx_v[pl.ds(t * L, L)] = idx_v[pl.ds(t * L, L)] + off

    FUSE1 = (AHEAD * CHUNK + L - 1) // L + 1

    @pl.loop(0, FUSE1, unroll=8)
    def _fuse_head(t):
        fuse(t)

    def gather(j, b):
        pltpu.async_copy(table_hbm.at[idx_v.at[pl.ds(j * CHUNK, CHUNK)]],
                         bufs[b], gsem[b])

    def wait_gather(b):
        pltpu.make_async_copy(table_hbm.at[pl.ds(0, CHUNK)], bufs[b],
                              gsem[b]).wait()

    def store(j, b):
        pltpu.async_copy(bufs[b], out_hbm.at[pl.ds(base + j * CHUNK, CHUNK)],
                         ssem[b])

    def wait_store(b):
        pltpu.make_async_copy(bufs[b], out_hbm.at[pl.ds(0, CHUNK)],
                              ssem[b]).wait()

    def step(j, b, prefetch, wait_prev):
        wait_gather(b)
        store(j, b)
        if prefetch:
            bn = (b + AHEAD) % NBUF
            if wait_prev:
                wait_store(bn)
            gather(j + AHEAD, bn)

    for b in range(AHEAD):
        gather(b, b)

    wait_units(isem, UPW - HEAD_UNITS)

    @pl.loop(FUSE1, PER_W // L, unroll=8)
    def _fuse_tail(t):
        fuse(t)

    for j in range(N_CHUNK):
        step(j, j % NBUF, j + AHEAD < N_CHUNK,
             j + AHEAD >= NBUF)
    for j in range(N_CHUNK - NBUF, N_CHUNK):
        wait_store(j % NBUF)


def kernel(inputs, table):
    out = _sok_gather(inputs.T, table)
    return out.reshape(NUM_FIELDS, BATCH, EMBED_DIM).transpose(1, 0, 2)
